# Initial kernel scaffold; baseline (speedup 1.0000x reference)
import functools

import jax
import jax.numpy as jnp
from jax import lax
from jax.experimental import pallas as pl
from jax.experimental.pallas import tpu as pltpu
from jax.experimental.pallas import tpu_sc as plsc

NUM_EMB = 1_000_000
DIM = 32
BATCH = 16384
N_FIELDS = 26
TOTAL = BATCH * N_FIELDS

NC = 2
NS = 16
NW = NC * NS
PER_TILE = TOTAL // NW
CHUNK = 128
NCHUNK = PER_TILE // CHUNK

_mesh = plsc.VectorSubcoreMesh(core_axis_name="c", subcore_axis_name="s")


@functools.partial(
    pl.kernel,
    mesh=_mesh,
    out_type=jax.ShapeDtypeStruct((TOTAL, DIM), jnp.float32),
    scratch_types=[
        pltpu.VMEM((NCHUNK, CHUNK), jnp.int32),
        pltpu.VMEM((CHUNK, DIM), jnp.float32),
        pltpu.SemaphoreType.DMA,
    ],
)
def _gather_sc(idx_hbm, table_hbm, out_hbm, idx_v, rows_v, sem):
    wid = lax.axis_index("s") * NC + lax.axis_index("c")
    base = wid * PER_TILE
    pltpu.sync_copy(idx_hbm.at[wid], idx_v)

    def body(j, carry):
        pltpu.async_copy(table_hbm.at[idx_v.at[j]], rows_v, sem).wait()
        pltpu.sync_copy(rows_v, out_hbm.at[pl.ds(base + j * CHUNK, CHUNK)])
        return carry

    lax.fori_loop(0, NCHUNK, body, 0)


def kernel(indices, table):
    idx = indices.astype(jnp.int32).reshape(NW, NCHUNK, CHUNK)
    out = _gather_sc(idx, table)
    return out.reshape(BATCH, N_FIELDS, DIM)


# speedup vs baseline: 1.4364x; 1.4364x over previous
import functools

import jax
import jax.numpy as jnp
from jax import lax
from jax.experimental import pallas as pl
from jax.experimental.pallas import tpu as pltpu
from jax.experimental.pallas import tpu_sc as plsc

NUM_EMB = 1_000_000
DIM = 32
BATCH = 16384
N_FIELDS = 26
TOTAL = BATCH * N_FIELDS

NC = 2
NS = 16
NW = NC * NS
PER_TILE = TOTAL // NW
CHUNK = 128
NCHUNK = PER_TILE // CHUNK

_mesh = plsc.VectorSubcoreMesh(core_axis_name="c", subcore_axis_name="s")


@functools.partial(
    pl.kernel,
    mesh=_mesh,
    out_type=jax.ShapeDtypeStruct((TOTAL, DIM), jnp.float32),
    compiler_params=pltpu.CompilerParams(use_tc_tiling_on_sc=False),
    scratch_types=[
        pltpu.VMEM((NCHUNK, CHUNK), jnp.int32),
        pltpu.VMEM((CHUNK, DIM), jnp.float32),
        pltpu.SemaphoreType.DMA,
    ],
)
def _gather_sc(idx_hbm, table_hbm, out_hbm, idx_v, rows_v, sem):
    wid = lax.axis_index("s") * NC + lax.axis_index("c")
    base = wid * PER_TILE
    pltpu.sync_copy(idx_hbm.at[wid], idx_v)

    def body(j, carry):
        pltpu.async_copy(table_hbm.at[idx_v.at[j]], rows_v, sem).wait()
        pltpu.sync_copy(rows_v, out_hbm.at[pl.ds(base + j * CHUNK, CHUNK)])
        return carry

    lax.fori_loop(0, NCHUNK, body, 0)


def kernel(indices, table):
    idx = indices.astype(jnp.int32).reshape(NW, NCHUNK, CHUNK)
    out = _gather_sc(idx, table)
    return out.reshape(BATCH, N_FIELDS, DIM)


# speedup vs baseline: 1.5743x; 1.0960x over previous
import functools

import jax
import jax.numpy as jnp
from jax import lax
from jax.experimental import pallas as pl
from jax.experimental.pallas import tpu as pltpu
from jax.experimental.pallas import tpu_sc as plsc

NUM_EMB = 1_000_000
DIM = 32
BATCH = 16384
N_FIELDS = 26
TOTAL = BATCH * N_FIELDS

NC = 2
NS = 16
NW = NC * NS
PER_TILE = TOTAL // NW
CHUNK = 128
NCHUNK = PER_TILE // CHUNK
G = 8
ROWS_G = G * CHUNK
NGROUP = NCHUNK // G

_mesh = plsc.VectorSubcoreMesh(core_axis_name="c", subcore_axis_name="s")


@functools.partial(
    pl.kernel,
    mesh=_mesh,
    out_type=jax.ShapeDtypeStruct((TOTAL, DIM), jnp.float32),
    compiler_params=pltpu.CompilerParams(use_tc_tiling_on_sc=False),
    scratch_types=[
        pltpu.VMEM((NCHUNK, CHUNK), jnp.int32),
        pltpu.VMEM((2, ROWS_G, DIM), jnp.float32),
        pltpu.SemaphoreType.DMA,
        pltpu.SemaphoreType.DMA,
        pltpu.SemaphoreType.DMA,
        pltpu.SemaphoreType.DMA,
    ],
)
def _gather_sc(idx_hbm, table_hbm, out_hbm, idx_v, buf, gsem0, gsem1,
               ssem0, ssem1):
    wid = lax.axis_index("s") * NC + lax.axis_index("c")
    base = wid * PER_TILE
    gsems = (gsem0, gsem1)
    ssems = (ssem0, ssem1)

    pltpu.sync_copy(idx_hbm.at[wid], idx_v)

    def start_gathers(g, h):
        def body(j, c):
            pltpu.async_copy(
                table_hbm.at[idx_v.at[g * G + j]],
                buf.at[h].at[pl.ds(j * CHUNK, CHUNK)],
                gsems[h],
            )
            return c
        lax.fori_loop(0, G, body, 0)

    def wait_gathers(h):
        pltpu.make_async_copy(out_hbm.at[pl.ds(0, ROWS_G)], buf.at[h],
                              gsems[h]).wait()

    def start_scatter(g, h):
        pltpu.async_copy(buf.at[h],
                         out_hbm.at[pl.ds(base + g * ROWS_G, ROWS_G)],
                         ssems[h])

    def wait_scatter(h):
        pltpu.make_async_copy(buf.at[h], out_hbm.at[pl.ds(base, ROWS_G)],
                              ssems[h]).wait()

    start_gathers(0, 0)
    for g in range(NGROUP):
        h = g % 2
        if g + 1 < NGROUP:
            if g >= 1:
                wait_scatter(1 - h)
            start_gathers(g + 1, 1 - h)
        wait_gathers(h)
        start_scatter(g, h)
    wait_scatter(0)
    wait_scatter(1)


def kernel(indices, table):
    idx = indices.astype(jnp.int32).reshape(NW, NCHUNK, CHUNK)
    out = _gather_sc(idx, table)
    return out.reshape(BATCH, N_FIELDS, DIM)


# speedup vs baseline: 1.5750x; 1.0005x over previous
import functools

import jax
import jax.numpy as jnp
from jax import lax
from jax.experimental import pallas as pl
from jax.experimental.pallas import tpu as pltpu
from jax.experimental.pallas import tpu_sc as plsc

NUM_EMB = 1_000_000
DIM = 32
BATCH = 16384
N_FIELDS = 26
TOTAL = BATCH * N_FIELDS

NC = 2
NS = 16
NW = NC * NS
PER_TILE = TOTAL // NW
CHUNK = 512
NCHUNK = PER_TILE // CHUNK
G = 2
ROWS_G = G * CHUNK
NGROUP = NCHUNK // G

_mesh = plsc.VectorSubcoreMesh(core_axis_name="c", subcore_axis_name="s")


@functools.partial(
    pl.kernel,
    mesh=_mesh,
    out_type=jax.ShapeDtypeStruct((TOTAL, DIM), jnp.float32),
    compiler_params=pltpu.CompilerParams(use_tc_tiling_on_sc=False),
    scratch_types=[
        pltpu.VMEM((NCHUNK, CHUNK), jnp.int32),
        pltpu.VMEM((2, ROWS_G, DIM), jnp.float32),
        pltpu.SemaphoreType.DMA,
        pltpu.SemaphoreType.DMA,
        pltpu.SemaphoreType.DMA,
        pltpu.SemaphoreType.DMA,
    ],
)
def _gather_sc(idx_hbm, table_hbm, out_hbm, idx_v, buf, gsem0, gsem1,
               ssem0, ssem1):
    wid = lax.axis_index("s") * NC + lax.axis_index("c")
    base = wid * PER_TILE
    gsems = (gsem0, gsem1)
    ssems = (ssem0, ssem1)

    pltpu.sync_copy(idx_hbm.at[wid], idx_v)

    def start_gathers(g, h):
        def body(j, c):
            pltpu.async_copy(
                table_hbm.at[idx_v.at[g * G + j]],
                buf.at[h].at[pl.ds(j * CHUNK, CHUNK)],
                gsems[h],
            )
            return c
        lax.fori_loop(0, G, body, 0)

    def wait_gathers(h):
        pltpu.make_async_copy(out_hbm.at[pl.ds(0, ROWS_G)], buf.at[h],
                              gsems[h]).wait()

    def start_scatter(g, h):
        pltpu.async_copy(buf.at[h],
                         out_hbm.at[pl.ds(base + g * ROWS_G, ROWS_G)],
                         ssems[h])

    def wait_scatter(h):
        pltpu.make_async_copy(buf.at[h], out_hbm.at[pl.ds(base, ROWS_G)],
                              ssems[h]).wait()

    start_gathers(0, 0)
    for g in range(NGROUP):
        h = g % 2
        if g + 1 < NGROUP:
            if g >= 1:
                wait_scatter(1 - h)
            start_gathers(g + 1, 1 - h)
        wait_gathers(h)
        start_scatter(g, h)
    wait_scatter(0)
    wait_scatter(1)


def kernel(indices, table):
    idx = indices.astype(jnp.int32).reshape(NW, NCHUNK, CHUNK)
    out = _gather_sc(idx, table)
    return out.reshape(BATCH, N_FIELDS, DIM)
